# baseline (speedup 1.0000x reference)
import functools

import jax
import jax.numpy as jnp
from jax import lax
from jax.experimental import pallas as pl
from jax.experimental.pallas import tpu as pltpu
from jax.experimental.pallas import tpu_sc as plsc

N = 10000
E = 320000
D = 128
DW = D // 2

NC = 2
NS = 16
NW = NC * NS
CHUNK = 48
GR = 6
SR = 3
GPD = 4
RPD = 2
NCHUNKS = 216
EPW = NCHUNKS * CHUNK
E_PAD = NW * EPW
ROWS_PER_S = N // NS
OUTER = NCHUNKS // GR


def _spmm_body(h_hbm, row_hbm, col_hbm, val_hbm, out_hbm,
               acc, colv, valv, rowv, gbuf, sbuf, gsem, rsem, vsem, ssem):
    cid = lax.axis_index("c")
    sid = lax.axis_index("s")
    w = cid * NS + sid
    ebase = w * EPW

    pltpu.sync_copy(col_hbm.at[pl.ds(ebase, EPW)], colv)

    def zero_body(e, _):
        for j in range(D // 16):
            sbuf[0, e, pl.ds(j * 16, 16)] = jnp.zeros((16,), jnp.float32)
        return 0

    lax.fori_loop(0, CHUNK, zero_body, 0)
    abase = sid * ROWS_PER_S
    for k in range(ROWS_PER_S // CHUNK):
        pltpu.sync_copy(sbuf.at[0], acc.at[pl.ds(abase + k * CHUNK, CHUNK)])
    rem = ROWS_PER_S % CHUNK
    pltpu.sync_copy(sbuf.at[0, pl.ds(0, rem)],
                    acc.at[pl.ds(abase + ROWS_PER_S - rem, rem)])

    for c in range(GPD):
        pltpu.async_copy(h_hbm.at[colv.at[pl.ds(c * CHUNK, CHUNK)]],
                         gbuf.at[c], gsem.at[c])
    for c in range(RPD):
        pltpu.async_copy(row_hbm.at[pl.ds(ebase + c * CHUNK, CHUNK)],
                         rowv.at[c], rsem.at[c])
        pltpu.async_copy(val_hbm.at[pl.ds(ebase + c * CHUNK, CHUNK)],
                         valv.at[c], vsem.at[c])

    plsc.subcore_barrier()

    def outer_body(o, _):
        for b in range(GR):
            s = o * GR + b
            sb = b % SR
            pltpu.make_async_copy(h_hbm.at[pl.ds(0, CHUNK)],
                                  gbuf.at[b], gsem.at[b]).wait()
            pltpu.make_async_copy(val_hbm.at[pl.ds(0, CHUNK)],
                                  valv.at[b], vsem.at[b]).wait()

            def sdrain(sb=sb):
                pltpu.make_async_copy(h_hbm.at[pl.ds(0, CHUNK)],
                                      sbuf.at[sb], ssem.at[sb]).wait()

            if b >= SR:
                sdrain()
            else:
                @pl.when(o > 0)
                def _(sdrain=sdrain):
                    sdrain()

            def scale_body(g, _, b=b, sb=sb):
                vv = valv[b, pl.ds(g * 16, 16)]
                for l in range(16):
                    v = vv[l]
                    e = g * 16 + l
                    for j in range(DW // 16):
                        wv = gbuf[b, e, pl.ds(j * 32, 32)]
                        lo, hi = plsc.unpack(
                            wv, format=plsc.PackFormat.INTERLEAVED)
                        sbuf[sb, e, pl.ds(j * 16, 16)] = lo * v
                        sbuf[sb, e, pl.ds(64 + j * 16, 16)] = hi * v
                return 0

            lax.fori_loop(0, CHUNK // 16, scale_body, 0)

            pltpu.make_async_copy(row_hbm.at[pl.ds(0, CHUNK)],
                                  rowv.at[b], rsem.at[b]).wait()
            pltpu.async_copy(sbuf.at[sb], acc.at[rowv.at[b]], ssem.at[sb],
                             add=True)

            def gissue(o=o, b=b):
                t = o * GR + b + GPD
                nb = (b + GPD) % GR
                pltpu.async_copy(h_hbm.at[colv.at[pl.ds(t * CHUNK, CHUNK)]],
                                 gbuf.at[nb], gsem.at[nb])

            def rissue(o=o, b=b):
                t = o * GR + b + RPD
                nb = (b + RPD) % GR
                pltpu.async_copy(row_hbm.at[pl.ds(ebase + t * CHUNK, CHUNK)],
                                 rowv.at[nb], rsem.at[nb])
                pltpu.async_copy(val_hbm.at[pl.ds(ebase + t * CHUNK, CHUNK)],
                                 valv.at[nb], vsem.at[nb])

            if b < GR - GPD:
                gissue()
            else:
                @pl.when(o < OUTER - 1)
                def _(gissue=gissue):
                    gissue()
            if b < GR - RPD:
                rissue()
            else:
                @pl.when(o < OUTER - 1)
                def _(rissue=rissue):
                    rissue()
        return 0

    lax.fori_loop(0, OUTER, outer_body, 0)

    for sb in range(SR):
        pltpu.make_async_copy(h_hbm.at[pl.ds(0, CHUNK)],
                              sbuf.at[sb], ssem.at[sb]).wait()

    plsc.subcore_barrier()

    off = pl.multiple_of(sid * 624, 8)
    pltpu.sync_copy(acc.at[pl.ds(off, 624)], out_hbm.at[cid, pl.ds(off, 624)])

    @pl.when(sid == 0)
    def _():
        pltpu.sync_copy(acc.at[pl.ds(NS * 624, N - NS * 624)],
                        out_hbm.at[cid, pl.ds(NS * 624, N - NS * 624)])


@jax.jit
def _spmm_sc(hpack, row, col, vals):
    mesh = plsc.VectorSubcoreMesh(core_axis_name="c", subcore_axis_name="s")
    return pl.kernel(
        _spmm_body,
        mesh=mesh,
        compiler_params=pltpu.CompilerParams(use_tc_tiling_on_sc=False,
                                             needs_layout_passes=False),
        out_type=jax.ShapeDtypeStruct((NC, N, D), jnp.float32),
        scratch_types=[
            pltpu.VMEM_SHARED((N, D), jnp.float32),
            pltpu.VMEM((EPW,), jnp.int32),
            pltpu.VMEM((GR, CHUNK), jnp.float32),
            pltpu.VMEM((GR, CHUNK), jnp.int32),
            pltpu.VMEM((GR, CHUNK, D), jnp.bfloat16),
            pltpu.VMEM((SR, CHUNK, D), jnp.float32),
            pltpu.SemaphoreType.DMA((GR,)),
            pltpu.SemaphoreType.DMA((GR,)),
            pltpu.SemaphoreType.DMA((GR,)),
            pltpu.SemaphoreType.DMA((SR,)),
        ],
    )(hpack, row, col, vals)


def _dense_body(h_ref, w_ref, b_ref, o_ref, *, act, sum2, to_bf16):
    h = h_ref[...]
    if sum2:
        h = h[0] + h[1]
    if act:
        h = jnp.maximum(h, 0.0)
    out = (jnp.dot(h, w_ref[...], preferred_element_type=jnp.float32)
           + b_ref[...])
    o_ref[...] = out.astype(jnp.bfloat16) if to_bf16 else out


def _dense_tc(h, w, b, act, sum2, to_bf16=False):
    n = h.shape[-2]
    dt = jnp.bfloat16 if to_bf16 else jnp.float32
    return pl.pallas_call(
        functools.partial(_dense_body, act=act, sum2=sum2, to_bf16=to_bf16),
        out_shape=jax.ShapeDtypeStruct((n, w.shape[1]), dt),
    )(h, w, b.reshape(1, -1))


def _pack_rows(h16):
    return jnp.stack([h16[:, :DW], h16[:, DW:]], axis=-1).reshape(N, D)


def kernel(x, edge_index, adj_values, W1, b1, W2, b2, Wc, bc):
    pad = E_PAD - E
    spread = (jnp.arange(pad, dtype=jnp.int32) * 13) % N
    row = jnp.concatenate([edge_index[0].astype(jnp.int32), spread])
    col = jnp.concatenate([edge_index[1].astype(jnp.int32), spread])
    vals = jnp.concatenate(
        [adj_values.astype(jnp.float32), jnp.zeros((pad,), jnp.float32)])
    h16 = _dense_tc(x, W1, b1, act=False, sum2=False, to_bf16=True)
    p = _spmm_sc(_pack_rows(h16), row, col, vals)
    h16b = _dense_tc(p, W2, b2, act=True, sum2=True, to_bf16=True)
    q = _spmm_sc(_pack_rows(h16b), row, col, vals)
    return _dense_tc(q, Wc, bc, act=False, sum2=True)


# speedup vs baseline: 2.1347x; 2.1347x over previous
import functools

import jax
import jax.numpy as jnp
from jax import lax
from jax.experimental import pallas as pl
from jax.experimental.pallas import tpu as pltpu
from jax.experimental.pallas import tpu_sc as plsc

N = 10000
E = 320000
D = 128

NC = 2
NS = 16
NW = NC * NS
CHUNK = 48
GR = 6
GPD = 4
RPD = 2
NCHUNKS = 216
EPW = NCHUNKS * CHUNK
E_PAD = NW * EPW
ROWS_PER_S = N // NS
OUTER = NCHUNKS // GR


def _spmm_body(h_hbm, row_hbm, col_hbm, val_hbm, out_hbm,
               acc, colv, valv, rowv, gbuf, gsem, rsem, vsem, ssem):
    cid = lax.axis_index("c")
    sid = lax.axis_index("s")
    w = cid * NS + sid
    ebase = w * EPW

    pltpu.sync_copy(col_hbm.at[pl.ds(ebase, EPW)], colv)

    def zero_body(e, _):
        for j in range(D // 16):
            gbuf[0, e, pl.ds(j * 16, 16)] = jnp.zeros((16,), jnp.float32)
        return 0

    lax.fori_loop(0, CHUNK, zero_body, 0)
    abase = sid * ROWS_PER_S
    for k in range(ROWS_PER_S // CHUNK):
        pltpu.sync_copy(gbuf.at[0], acc.at[pl.ds(abase + k * CHUNK, CHUNK)])
    rem = ROWS_PER_S % CHUNK
    pltpu.sync_copy(gbuf.at[0, pl.ds(0, rem)],
                    acc.at[pl.ds(abase + ROWS_PER_S - rem, rem)])

    for c in range(GPD):
        pltpu.async_copy(h_hbm.at[colv.at[pl.ds(c * CHUNK, CHUNK)]],
                         gbuf.at[c], gsem.at[c])
    for c in range(RPD):
        pltpu.async_copy(row_hbm.at[pl.ds(ebase + c * CHUNK, CHUNK)],
                         rowv.at[c], rsem.at[c])
        pltpu.async_copy(val_hbm.at[pl.ds(ebase + c * CHUNK, CHUNK)],
                         valv.at[c], vsem.at[c])

    plsc.subcore_barrier()

    def outer_body(o, _):
        for b in range(GR):
            s = o * GR + b
            pltpu.make_async_copy(h_hbm.at[pl.ds(0, CHUNK)],
                                  gbuf.at[b], gsem.at[b]).wait()
            pltpu.make_async_copy(val_hbm.at[pl.ds(0, CHUNK)],
                                  valv.at[b], vsem.at[b]).wait()

            def scale_body(g, _, b=b):
                vv = valv[b, pl.ds(g * 16, 16)]
                for l in range(16):
                    v = vv[l]
                    e = g * 16 + l
                    for j in range(D // 16):
                        gbuf[b, e, pl.ds(j * 16, 16)] = (
                            gbuf[b, e, pl.ds(j * 16, 16)] * v)
                return 0

            lax.fori_loop(0, CHUNK // 16, scale_body, 0)

            pltpu.make_async_copy(row_hbm.at[pl.ds(0, CHUNK)],
                                  rowv.at[b], rsem.at[b]).wait()
            pltpu.async_copy(gbuf.at[b], acc.at[rowv.at[b]], ssem.at[b],
                             add=True)

            nb = (b + GPD) % GR

            def gissue(o=o, b=b, nb=nb):
                t = o * GR + b + GPD
                pltpu.async_copy(h_hbm.at[colv.at[pl.ds(t * CHUNK, CHUNK)]],
                                 gbuf.at[nb], gsem.at[nb])

            def sdrain(nb=nb):
                pltpu.make_async_copy(h_hbm.at[pl.ds(0, CHUNK)],
                                      gbuf.at[nb], ssem.at[nb]).wait()

            def rissue(o=o, b=b):
                t = o * GR + b + RPD
                nr = (b + RPD) % GR
                pltpu.async_copy(row_hbm.at[pl.ds(ebase + t * CHUNK, CHUNK)],
                                 rowv.at[nr], rsem.at[nr])
                pltpu.async_copy(val_hbm.at[pl.ds(ebase + t * CHUNK, CHUNK)],
                                 valv.at[nr], vsem.at[nr])

            if b < GR - GPD:
                @pl.when(o > 0)
                def _(gissue=gissue, sdrain=sdrain):
                    sdrain()
                    gissue()

                @pl.when(o == 0)
                def _(gissue=gissue):
                    gissue()
            else:
                @pl.when(o < OUTER - 1)
                def _(gissue=gissue, sdrain=sdrain):
                    sdrain()
                    gissue()
            if b < GR - RPD:
                rissue()
            else:
                @pl.when(o < OUTER - 1)
                def _(rissue=rissue):
                    rissue()
        return 0

    lax.fori_loop(0, OUTER, outer_body, 0)

    for b in range(GR):
        pltpu.make_async_copy(h_hbm.at[pl.ds(0, CHUNK)],
                              gbuf.at[b], ssem.at[b]).wait()

    plsc.subcore_barrier()

    off = pl.multiple_of(sid * 624, 8)
    pltpu.sync_copy(acc.at[pl.ds(off, 624)], out_hbm.at[cid, pl.ds(off, 624)])

    @pl.when(sid == 0)
    def _():
        pltpu.sync_copy(acc.at[pl.ds(NS * 624, N - NS * 624)],
                        out_hbm.at[cid, pl.ds(NS * 624, N - NS * 624)])


@jax.jit
def _spmm_sc(h, row, col, vals):
    mesh = plsc.VectorSubcoreMesh(core_axis_name="c", subcore_axis_name="s")
    return pl.kernel(
        _spmm_body,
        mesh=mesh,
        compiler_params=pltpu.CompilerParams(use_tc_tiling_on_sc=False),
        out_type=jax.ShapeDtypeStruct((NC, N, D), jnp.float32),
        scratch_types=[
            pltpu.VMEM_SHARED((N, D), jnp.float32),
            pltpu.VMEM((EPW,), jnp.int32),
            pltpu.VMEM((GR, CHUNK), jnp.float32),
            pltpu.VMEM((GR, CHUNK), jnp.int32),
            pltpu.VMEM((GR, CHUNK, D), jnp.float32),
            pltpu.SemaphoreType.DMA((GR,)),
            pltpu.SemaphoreType.DMA((GR,)),
            pltpu.SemaphoreType.DMA((GR,)),
            pltpu.SemaphoreType.DMA((GR,)),
        ],
    )(h, row, col, vals)


def _dense_body(h_ref, w_ref, b_ref, o_ref, *, act, sum2):
    h = h_ref[...]
    if sum2:
        h = h[0] + h[1]
    if act:
        h = jnp.maximum(h, 0.0)
    o_ref[...] = (jnp.dot(h, w_ref[...], preferred_element_type=jnp.float32)
                  + b_ref[...])


def _dense_tc(h, w, b, act, sum2):
    n = h.shape[-2]
    return pl.pallas_call(
        functools.partial(_dense_body, act=act, sum2=sum2),
        out_shape=jax.ShapeDtypeStruct((n, w.shape[1]), jnp.float32),
    )(h, w, b.reshape(1, -1))


def kernel(x, edge_index, adj_values, W1, b1, W2, b2, Wc, bc):
    pad = E_PAD - E
    spread = (jnp.arange(pad, dtype=jnp.int32) * 13) % N
    row = jnp.concatenate([edge_index[0].astype(jnp.int32), spread])
    col = jnp.concatenate([edge_index[1].astype(jnp.int32), spread])
    vals = jnp.concatenate(
        [adj_values.astype(jnp.float32), jnp.zeros((pad,), jnp.float32)])
    h = _dense_tc(x, W1, b1, act=False, sum2=False)
    p = _spmm_sc(h, row, col, vals)
    h2 = _dense_tc(p, W2, b2, act=True, sum2=True)
    q = _spmm_sc(h2, row, col, vals)
    return _dense_tc(q, Wc, bc, act=False, sum2=True)


# speedup vs baseline: 2.5105x; 1.1760x over previous
import functools

import jax
import jax.numpy as jnp
from jax import lax
from jax.experimental import pallas as pl
from jax.experimental.pallas import tpu as pltpu
from jax.experimental.pallas import tpu_sc as plsc

N = 10000
E = 320000
D = 128
DC = 48
NCLS = 40

NC = 2
NS = 16
NW = NC * NS
CHUNK = 48
GR = 6
GPD = 4
RPD = 2
NCHUNKS = 216
EPW = NCHUNKS * CHUNK
E_PAD = NW * EPW
ROWS_PER_S = N // NS
OUTER = NCHUNKS // GR


def _spmm_body(h_hbm, row_hbm, col_hbm, val_hbm, out_hbm,
               acc, colv, valv, rowv, gbuf, gsem, rsem, vsem, ssem, *, dd):
    cid = lax.axis_index("c")
    sid = lax.axis_index("s")
    w = cid * NS + sid
    ebase = w * EPW

    pltpu.sync_copy(col_hbm.at[pl.ds(ebase, EPW)], colv)

    def zero_body(e, _):
        for j in range(dd // 16):
            gbuf[0, e, pl.ds(j * 16, 16)] = jnp.zeros((16,), jnp.float32)
        return 0

    lax.fori_loop(0, CHUNK, zero_body, 0)
    abase = sid * ROWS_PER_S
    for k in range(ROWS_PER_S // CHUNK):
        pltpu.sync_copy(gbuf.at[0], acc.at[pl.ds(abase + k * CHUNK, CHUNK)])
    rem = ROWS_PER_S % CHUNK
    pltpu.sync_copy(gbuf.at[0, pl.ds(0, rem)],
                    acc.at[pl.ds(abase + ROWS_PER_S - rem, rem)])

    for c in range(GPD):
        pltpu.async_copy(h_hbm.at[colv.at[pl.ds(c * CHUNK, CHUNK)]],
                         gbuf.at[c], gsem.at[c])
    for c in range(RPD):
        pltpu.async_copy(row_hbm.at[pl.ds(ebase + c * CHUNK, CHUNK)],
                         rowv.at[c], rsem.at[c])
        pltpu.async_copy(val_hbm.at[pl.ds(ebase + c * CHUNK, CHUNK)],
                         valv.at[c], vsem.at[c])

    plsc.subcore_barrier()

    def outer_body(o, _):
        for b in range(GR):
            s = o * GR + b
            pltpu.make_async_copy(h_hbm.at[pl.ds(0, CHUNK)],
                                  gbuf.at[b], gsem.at[b]).wait()
            pltpu.make_async_copy(val_hbm.at[pl.ds(0, CHUNK)],
                                  valv.at[b], vsem.at[b]).wait()

            def scale_body(g, _, b=b):
                vv = valv[b, pl.ds(g * 16, 16)]
                for l in range(16):
                    v = vv[l]
                    e = g * 16 + l
                    for j in range(dd // 16):
                        gbuf[b, e, pl.ds(j * 16, 16)] = (
                            gbuf[b, e, pl.ds(j * 16, 16)] * v)
                return 0

            lax.fori_loop(0, CHUNK // 16, scale_body, 0)

            pltpu.make_async_copy(row_hbm.at[pl.ds(0, CHUNK)],
                                  rowv.at[b], rsem.at[b]).wait()
            pltpu.async_copy(gbuf.at[b], acc.at[rowv.at[b]], ssem.at[b],
                             add=True)

            nb = (b + GPD) % GR

            def gissue(o=o, b=b, nb=nb):
                t = o * GR + b + GPD
                pltpu.async_copy(h_hbm.at[colv.at[pl.ds(t * CHUNK, CHUNK)]],
                                 gbuf.at[nb], gsem.at[nb])

            def sdrain(nb=nb):
                pltpu.make_async_copy(h_hbm.at[pl.ds(0, CHUNK)],
                                      gbuf.at[nb], ssem.at[nb]).wait()

            def rissue(o=o, b=b):
                t = o * GR + b + RPD
                nr = (b + RPD) % GR
                pltpu.async_copy(row_hbm.at[pl.ds(ebase + t * CHUNK, CHUNK)],
                                 rowv.at[nr], rsem.at[nr])
                pltpu.async_copy(val_hbm.at[pl.ds(ebase + t * CHUNK, CHUNK)],
                                 valv.at[nr], vsem.at[nr])

            if b < GR - GPD:
                @pl.when(o > 0)
                def _(gissue=gissue, sdrain=sdrain):
                    sdrain()
                    gissue()

                @pl.when(o == 0)
                def _(gissue=gissue):
                    gissue()
            else:
                @pl.when(o < OUTER - 1)
                def _(gissue=gissue, sdrain=sdrain):
                    sdrain()
                    gissue()
            if b < GR - RPD:
                rissue()
            else:
                @pl.when(o < OUTER - 1)
                def _(rissue=rissue):
                    rissue()
        return 0

    lax.fori_loop(0, OUTER, outer_body, 0)

    for b in range(GR):
        pltpu.make_async_copy(h_hbm.at[pl.ds(0, CHUNK)],
                              gbuf.at[b], ssem.at[b]).wait()

    plsc.subcore_barrier()

    off = pl.multiple_of(sid * 624, 8)
    pltpu.sync_copy(acc.at[pl.ds(off, 624)], out_hbm.at[cid, pl.ds(off, 624)])

    @pl.when(sid == 0)
    def _():
        pltpu.sync_copy(acc.at[pl.ds(NS * 624, N - NS * 624)],
                        out_hbm.at[cid, pl.ds(NS * 624, N - NS * 624)])


def _make_spmm(dd):
    mesh = plsc.VectorSubcoreMesh(core_axis_name="c", subcore_axis_name="s")
    return pl.kernel(
        functools.partial(_spmm_body, dd=dd),
        mesh=mesh,
        compiler_params=pltpu.CompilerParams(use_tc_tiling_on_sc=False),
        out_type=jax.ShapeDtypeStruct((NC, N, dd), jnp.float32),
        scratch_types=[
            pltpu.VMEM_SHARED((N, dd), jnp.float32),
            pltpu.VMEM((EPW,), jnp.int32),
            pltpu.VMEM((GR, CHUNK), jnp.float32),
            pltpu.VMEM((GR, CHUNK), jnp.int32),
            pltpu.VMEM((GR, CHUNK, dd), jnp.float32),
            pltpu.SemaphoreType.DMA((GR,)),
            pltpu.SemaphoreType.DMA((GR,)),
            pltpu.SemaphoreType.DMA((GR,)),
            pltpu.SemaphoreType.DMA((GR,)),
        ],
    )


def _dense1_body(x_ref, w_ref, b_ref, o_ref):
    o_ref[...] = (jnp.dot(x_ref[...], w_ref[...],
                          preferred_element_type=jnp.float32) + b_ref[...])


def _dense2_body(p_ref, w2_ref, b2_ref, wc_ref, o_ref):
    t = jnp.maximum(p_ref[0] + p_ref[1], 0.0)
    h2 = (jnp.dot(t, w2_ref[...], preferred_element_type=jnp.float32)
          + b2_ref[...])
    o_ref[...] = jnp.dot(h2, wc_ref[...], preferred_element_type=jnp.float32)


def _dense3_body(q_ref, bc_ref, o_ref):
    o_ref[...] = (q_ref[0] + q_ref[1])[:, :NCLS] + bc_ref[...]


def kernel(x, edge_index, adj_values, W1, b1, W2, b2, Wc, bc):
    pad = E_PAD - E
    spread = (jnp.arange(pad, dtype=jnp.int32) * 13) % N
    row = jnp.concatenate([edge_index[0].astype(jnp.int32), spread])
    col = jnp.concatenate([edge_index[1].astype(jnp.int32), spread])
    vals = jnp.concatenate(
        [adj_values.astype(jnp.float32), jnp.zeros((pad,), jnp.float32)])

    h = pl.pallas_call(
        _dense1_body,
        out_shape=jax.ShapeDtypeStruct((N, D), jnp.float32),
    )(x, W1, b1.reshape(1, -1))

    p = _make_spmm(D)(h, row, col, vals)

    Wcp = jnp.pad(Wc, ((0, 0), (0, DC - NCLS)))
    z = pl.pallas_call(
        _dense2_body,
        out_shape=jax.ShapeDtypeStruct((N, DC), jnp.float32),
    )(p, W2, b2.reshape(1, -1), Wcp)

    q = _make_spmm(DC)(z, row, col, vals)

    return pl.pallas_call(
        _dense3_body,
        out_shape=jax.ShapeDtypeStruct((N, NCLS), jnp.float32),
    )(q, bc.reshape(1, -1))


# speedup vs baseline: 2.5241x; 1.0054x over previous
import functools

import jax
import jax.numpy as jnp
from jax import lax
from jax.experimental import pallas as pl
from jax.experimental.pallas import tpu as pltpu
from jax.experimental.pallas import tpu_sc as plsc

N = 10000
E = 320000
D = 128
DC = 48
NCLS = 40

NC = 2
NS = 16
NW = NC * NS
CHUNK = 48
GR = 6
GPD = 4
RPD = 2
NCHUNKS = 216
EPW = NCHUNKS * CHUNK
E_PAD = NW * EPW
ROWS_PER_S = N // NS
OUTER = NCHUNKS // GR


def _spmm_body(h_hbm, row_hbm, col_hbm, val_hbm, out_hbm,
               acc, colv, valv, rowv, gbuf, gsem, rsem, vsem, ssem, *, dd):
    cid = lax.axis_index("c")
    sid = lax.axis_index("s")
    w = cid * NS + sid
    ebase = w * EPW

    pltpu.async_copy(col_hbm.at[pl.ds(ebase, EPW)], colv, gsem.at[1])

    def zero_body(e, _):
        for j in range(dd // 16):
            gbuf[0, e, pl.ds(j * 16, 16)] = jnp.zeros((16,), jnp.float32)
        return 0

    lax.fori_loop(0, CHUNK, zero_body, 0)
    abase = sid * ROWS_PER_S
    rem = ROWS_PER_S % CHUNK
    for k in range(ROWS_PER_S // CHUNK):
        pltpu.async_copy(gbuf.at[0], acc.at[pl.ds(abase + k * CHUNK, CHUNK)],
                         gsem.at[0])
    pltpu.async_copy(gbuf.at[0, pl.ds(0, rem)],
                     acc.at[pl.ds(abase + ROWS_PER_S - rem, rem)], gsem.at[0])
    for k in range(ROWS_PER_S // CHUNK):
        pltpu.make_async_copy(gbuf.at[0],
                              acc.at[pl.ds(abase + k * CHUNK, CHUNK)],
                              gsem.at[0]).wait()
    pltpu.make_async_copy(gbuf.at[0, pl.ds(0, rem)],
                          acc.at[pl.ds(abase + ROWS_PER_S - rem, rem)],
                          gsem.at[0]).wait()
    pltpu.make_async_copy(col_hbm.at[pl.ds(ebase, EPW)], colv,
                          gsem.at[1]).wait()

    for c in range(GPD):
        pltpu.async_copy(h_hbm.at[colv.at[pl.ds(c * CHUNK, CHUNK)]],
                         gbuf.at[c], gsem.at[c])
    for c in range(RPD):
        pltpu.async_copy(row_hbm.at[pl.ds(ebase + c * CHUNK, CHUNK)],
                         rowv.at[c], rsem.at[c])
        pltpu.async_copy(val_hbm.at[pl.ds(ebase + c * CHUNK, CHUNK)],
                         valv.at[c], vsem.at[c])

    plsc.subcore_barrier()

    def outer_body(o, _):
        for b in range(GR):
            s = o * GR + b
            pltpu.make_async_copy(h_hbm.at[pl.ds(0, CHUNK)],
                                  gbuf.at[b], gsem.at[b]).wait()
            pltpu.make_async_copy(val_hbm.at[pl.ds(0, CHUNK)],
                                  valv.at[b], vsem.at[b]).wait()

            def scale_body(g, _, b=b):
                vv = valv[b, pl.ds(g * 16, 16)]
                for l in range(16):
                    v = vv[l]
                    e = g * 16 + l
                    for j in range(dd // 16):
                        gbuf[b, e, pl.ds(j * 16, 16)] = (
                            gbuf[b, e, pl.ds(j * 16, 16)] * v)
                return 0

            lax.fori_loop(0, CHUNK // 16, scale_body, 0)

            pltpu.make_async_copy(row_hbm.at[pl.ds(0, CHUNK)],
                                  rowv.at[b], rsem.at[b]).wait()
            pltpu.async_copy(gbuf.at[b], acc.at[rowv.at[b]], ssem.at[b],
                             add=True)

            nb = (b + GPD) % GR

            def gissue(o=o, b=b, nb=nb):
                t = o * GR + b + GPD
                pltpu.async_copy(h_hbm.at[colv.at[pl.ds(t * CHUNK, CHUNK)]],
                                 gbuf.at[nb], gsem.at[nb])

            def sdrain(nb=nb):
                pltpu.make_async_copy(h_hbm.at[pl.ds(0, CHUNK)],
                                      gbuf.at[nb], ssem.at[nb]).wait()

            def rissue(o=o, b=b):
                t = o * GR + b + RPD
                nr = (b + RPD) % GR
                pltpu.async_copy(row_hbm.at[pl.ds(ebase + t * CHUNK, CHUNK)],
                                 rowv.at[nr], rsem.at[nr])
                pltpu.async_copy(val_hbm.at[pl.ds(ebase + t * CHUNK, CHUNK)],
                                 valv.at[nr], vsem.at[nr])

            if b < GR - GPD:
                @pl.when(o > 0)
                def _(gissue=gissue, sdrain=sdrain):
                    sdrain()
                    gissue()

                @pl.when(o == 0)
                def _(gissue=gissue):
                    gissue()
            else:
                @pl.when(o < OUTER - 1)
                def _(gissue=gissue, sdrain=sdrain):
                    sdrain()
                    gissue()
            if b < GR - RPD:
                rissue()
            else:
                @pl.when(o < OUTER - 1)
                def _(rissue=rissue):
                    rissue()
        return 0

    lax.fori_loop(0, OUTER, outer_body, 0)

    for b in range(GR):
        pltpu.make_async_copy(h_hbm.at[pl.ds(0, CHUNK)],
                              gbuf.at[b], ssem.at[b]).wait()

    plsc.subcore_barrier()

    off = pl.multiple_of(sid * 624, 8)
    pltpu.sync_copy(acc.at[pl.ds(off, 624)], out_hbm.at[cid, pl.ds(off, 624)])

    @pl.when(sid == 0)
    def _():
        pltpu.sync_copy(acc.at[pl.ds(NS * 624, N - NS * 624)],
                        out_hbm.at[cid, pl.ds(NS * 624, N - NS * 624)])


def _make_spmm(dd):
    mesh = plsc.VectorSubcoreMesh(core_axis_name="c", subcore_axis_name="s")
    return pl.kernel(
        functools.partial(_spmm_body, dd=dd),
        mesh=mesh,
        compiler_params=pltpu.CompilerParams(use_tc_tiling_on_sc=False),
        out_type=jax.ShapeDtypeStruct((NC, N, dd), jnp.float32),
        scratch_types=[
            pltpu.VMEM_SHARED((N, dd), jnp.float32),
            pltpu.VMEM((EPW,), jnp.int32),
            pltpu.VMEM((GR, CHUNK), jnp.float32),
            pltpu.VMEM((GR, CHUNK), jnp.int32),
            pltpu.VMEM((GR, CHUNK, dd), jnp.float32),
            pltpu.SemaphoreType.DMA((GR,)),
            pltpu.SemaphoreType.DMA((GR,)),
            pltpu.SemaphoreType.DMA((GR,)),
            pltpu.SemaphoreType.DMA((GR,)),
        ],
    )


def _dense1_body(x_ref, w_ref, b_ref, o_ref):
    o_ref[...] = (jnp.dot(x_ref[...], w_ref[...],
                          preferred_element_type=jnp.float32) + b_ref[...])


def _dense2_body(p_ref, w2_ref, b2_ref, wc_ref, o_ref):
    t = jnp.maximum(p_ref[0] + p_ref[1], 0.0)
    h2 = (jnp.dot(t, w2_ref[...], preferred_element_type=jnp.float32)
          + b2_ref[...])
    o_ref[...] = jnp.dot(h2, wc_ref[...], preferred_element_type=jnp.float32)


def _dense3_body(q_ref, bc_ref, o_ref):
    o_ref[...] = (q_ref[0] + q_ref[1])[:, :NCLS] + bc_ref[...]


def kernel(x, edge_index, adj_values, W1, b1, W2, b2, Wc, bc):
    pad = E_PAD - E
    spread = (jnp.arange(pad, dtype=jnp.int32) * 13) % N
    row = jnp.concatenate([edge_index[0].astype(jnp.int32), spread])
    col = jnp.concatenate([edge_index[1].astype(jnp.int32), spread])
    vals = jnp.concatenate(
        [adj_values.astype(jnp.float32), jnp.zeros((pad,), jnp.float32)])

    h = pl.pallas_call(
        _dense1_body,
        out_shape=jax.ShapeDtypeStruct((N, D), jnp.float32),
    )(x, W1, b1.reshape(1, -1))

    p = _make_spmm(D)(h, row, col, vals)

    Wcp = jnp.pad(Wc, ((0, 0), (0, DC - NCLS)))
    z = pl.pallas_call(
        _dense2_body,
        out_shape=jax.ShapeDtypeStruct((N, DC), jnp.float32),
    )(p, W2, b2.reshape(1, -1), Wcp)

    q = _make_spmm(DC)(z, row, col, vals)

    return pl.pallas_call(
        _dense3_body,
        out_shape=jax.ShapeDtypeStruct((N, NCLS), jnp.float32),
    )(q, bc.reshape(1, -1))


# speedup vs baseline: 2.7364x; 1.0841x over previous
import functools

import jax
import jax.numpy as jnp
from jax import lax
from jax.experimental import pallas as pl
from jax.experimental.pallas import tpu as pltpu
from jax.experimental.pallas import tpu_sc as plsc

N = 10000
E = 320000
D = 128
DC = 48
NCLS = 40

NC = 2
NS = 16
NW = NC * NS
CHUNK = 48
GR = 6
GPD = 4
RPD = 2
CHUNK2 = 96
NCHUNKS = 216
EPW = NCHUNKS * CHUNK
E_PAD = NW * EPW
ROWS_PER_S = N // NS
OUTER = NCHUNKS // GR


def _spmm_body(h_hbm, row_hbm, col_hbm, val_hbm, out_hbm,
               acc, colv, valv, rowv, gbuf, gsem, rsem, vsem, ssem,
               *, dd, ck, outer):
    cid = lax.axis_index("c")
    sid = lax.axis_index("s")
    w = cid * NS + sid
    ebase = w * EPW

    pltpu.async_copy(col_hbm.at[pl.ds(ebase, EPW)], colv, gsem.at[1])

    def zero_body(e, _):
        for j in range(dd // 16):
            gbuf[0, e, pl.ds(j * 16, 16)] = jnp.zeros((16,), jnp.float32)
        return 0

    lax.fori_loop(0, ck, zero_body, 0)
    abase = sid * ROWS_PER_S
    rem = ROWS_PER_S % ck
    for k in range(ROWS_PER_S // ck):
        pltpu.async_copy(gbuf.at[0], acc.at[pl.ds(abase + k * ck, ck)],
                         gsem.at[0])
    pltpu.async_copy(gbuf.at[0, pl.ds(0, rem)],
                     acc.at[pl.ds(abase + ROWS_PER_S - rem, rem)], gsem.at[0])
    for k in range(ROWS_PER_S // ck):
        pltpu.make_async_copy(gbuf.at[0],
                              acc.at[pl.ds(abase + k * ck, ck)],
                              gsem.at[0]).wait()
    pltpu.make_async_copy(gbuf.at[0, pl.ds(0, rem)],
                          acc.at[pl.ds(abase + ROWS_PER_S - rem, rem)],
                          gsem.at[0]).wait()
    pltpu.make_async_copy(col_hbm.at[pl.ds(ebase, EPW)], colv,
                          gsem.at[1]).wait()

    for c in range(GPD):
        pltpu.async_copy(h_hbm.at[colv.at[pl.ds(c * ck, ck)]],
                         gbuf.at[c], gsem.at[c])
    for c in range(RPD):
        pltpu.async_copy(row_hbm.at[pl.ds(ebase + c * ck, ck)],
                         rowv.at[c], rsem.at[c])
        pltpu.async_copy(val_hbm.at[pl.ds(ebase + c * ck, ck)],
                         valv.at[c], vsem.at[c])

    plsc.subcore_barrier()

    def outer_body(o, _):
        for b in range(GR):
            s = o * GR + b
            pltpu.make_async_copy(h_hbm.at[pl.ds(0, ck)],
                                  gbuf.at[b], gsem.at[b]).wait()
            pltpu.make_async_copy(val_hbm.at[pl.ds(0, ck)],
                                  valv.at[b], vsem.at[b]).wait()

            def scale_body(g, _, b=b):
                vv = valv[b, pl.ds(g * 16, 16)]
                for l in range(16):
                    v = vv[l]
                    e = g * 16 + l
                    for j in range(dd // 16):
                        gbuf[b, e, pl.ds(j * 16, 16)] = (
                            gbuf[b, e, pl.ds(j * 16, 16)] * v)
                return 0

            lax.fori_loop(0, ck // 16, scale_body, 0)

            pltpu.make_async_copy(row_hbm.at[pl.ds(0, ck)],
                                  rowv.at[b], rsem.at[b]).wait()
            pltpu.async_copy(gbuf.at[b], acc.at[rowv.at[b]], ssem.at[b],
                             add=True)

            nb = (b + GPD) % GR

            def gissue(o=o, b=b, nb=nb):
                t = o * GR + b + GPD
                pltpu.async_copy(h_hbm.at[colv.at[pl.ds(t * ck, ck)]],
                                 gbuf.at[nb], gsem.at[nb])

            def sdrain(nb=nb):
                pltpu.make_async_copy(h_hbm.at[pl.ds(0, ck)],
                                      gbuf.at[nb], ssem.at[nb]).wait()

            def rissue(o=o, b=b):
                t = o * GR + b + RPD
                nr = (b + RPD) % GR
                pltpu.async_copy(row_hbm.at[pl.ds(ebase + t * ck, ck)],
                                 rowv.at[nr], rsem.at[nr])
                pltpu.async_copy(val_hbm.at[pl.ds(ebase + t * ck, ck)],
                                 valv.at[nr], vsem.at[nr])

            if b < GR - GPD:
                @pl.when(o > 0)
                def _(gissue=gissue, sdrain=sdrain):
                    sdrain()
                    gissue()

                @pl.when(o == 0)
                def _(gissue=gissue):
                    gissue()
            else:
                @pl.when(o < outer - 1)
                def _(gissue=gissue, sdrain=sdrain):
                    sdrain()
                    gissue()
            if b < GR - RPD:
                rissue()
            else:
                @pl.when(o < outer - 1)
                def _(rissue=rissue):
                    rissue()
        return 0

    lax.fori_loop(0, outer, outer_body, 0)

    for b in range(GR):
        pltpu.make_async_copy(h_hbm.at[pl.ds(0, ck)],
                              gbuf.at[b], ssem.at[b]).wait()

    plsc.subcore_barrier()

    off = pl.multiple_of(sid * 624, 8)
    pltpu.sync_copy(acc.at[pl.ds(off, 624)], out_hbm.at[cid, pl.ds(off, 624)])

    @pl.when(sid == 0)
    def _():
        pltpu.sync_copy(acc.at[pl.ds(NS * 624, N - NS * 624)],
                        out_hbm.at[cid, pl.ds(NS * 624, N - NS * 624)])


def _make_spmm(dd, ck):
    outer = EPW // ck // GR
    mesh = plsc.VectorSubcoreMesh(core_axis_name="c", subcore_axis_name="s")
    return pl.kernel(
        functools.partial(_spmm_body, dd=dd, ck=ck, outer=outer),
        mesh=mesh,
        compiler_params=pltpu.CompilerParams(use_tc_tiling_on_sc=False),
        out_type=jax.ShapeDtypeStruct((NC, N, dd), jnp.float32),
        scratch_types=[
            pltpu.VMEM_SHARED((N, dd), jnp.float32),
            pltpu.VMEM((EPW,), jnp.int32),
            pltpu.VMEM((GR, ck), jnp.float32),
            pltpu.VMEM((GR, ck), jnp.int32),
            pltpu.VMEM((GR, ck, dd), jnp.float32),
            pltpu.SemaphoreType.DMA((GR,)),
            pltpu.SemaphoreType.DMA((GR,)),
            pltpu.SemaphoreType.DMA((GR,)),
            pltpu.SemaphoreType.DMA((GR,)),
        ],
    )


def _dense1_body(x_ref, w_ref, b_ref, o_ref):
    o_ref[...] = (jnp.dot(x_ref[...], w_ref[...],
                          preferred_element_type=jnp.float32) + b_ref[...])


def _dense2_body(p_ref, w2_ref, b2_ref, wc_ref, o_ref):
    t = jnp.maximum(p_ref[0] + p_ref[1], 0.0)
    h2 = (jnp.dot(t, w2_ref[...], preferred_element_type=jnp.float32)
          + b2_ref[...])
    o_ref[...] = jnp.dot(h2, wc_ref[...], preferred_element_type=jnp.float32)


def _dense3_body(q_ref, bc_ref, o_ref):
    o_ref[...] = (q_ref[0] + q_ref[1])[:, :NCLS] + bc_ref[...]


def kernel(x, edge_index, adj_values, W1, b1, W2, b2, Wc, bc):
    pad = E_PAD - E
    spread = (jnp.arange(pad, dtype=jnp.int32) * 13) % N
    row = jnp.concatenate([edge_index[0].astype(jnp.int32), spread])
    col = jnp.concatenate([edge_index[1].astype(jnp.int32), spread])
    vals = jnp.concatenate(
        [adj_values.astype(jnp.float32), jnp.zeros((pad,), jnp.float32)])

    h = pl.pallas_call(
        _dense1_body,
        out_shape=jax.ShapeDtypeStruct((N, D), jnp.float32),
    )(x, W1, b1.reshape(1, -1))

    p = _make_spmm(D, CHUNK)(h, row, col, vals)

    Wcp = jnp.pad(Wc, ((0, 0), (0, DC - NCLS)))
    z = pl.pallas_call(
        _dense2_body,
        out_shape=jax.ShapeDtypeStruct((N, DC), jnp.float32),
    )(p, W2, b2.reshape(1, -1), Wcp)

    q = _make_spmm(DC, CHUNK2)(z, row, col, vals)

    return pl.pallas_call(
        _dense3_body,
        out_shape=jax.ShapeDtypeStruct((N, NCLS), jnp.float32),
    )(q, bc.reshape(1, -1))


# speedup vs baseline: 2.8233x; 1.0318x over previous
import functools

import jax
import jax.numpy as jnp
from jax import lax
from jax.experimental import pallas as pl
from jax.experimental.pallas import tpu as pltpu
from jax.experimental.pallas import tpu_sc as plsc

N = 10000
E = 320000
D = 128
DC = 48
NCLS = 40

NC = 2
NS = 16
NW = NC * NS
CHUNK = 48
GR = 6
GPD = 4
RPD = 2
CHUNK2 = 96
NCHUNKS = 216
EPW = NCHUNKS * CHUNK
E_PAD = NW * EPW
ROWS_PER_S = N // NS
OUTER = NCHUNKS // GR


def _spmm_body(h_hbm, row_hbm, col_hbm, val_hbm, out_hbm,
               acc, colv, valv, rowv, gbuf, gsem, rsem, vsem, ssem, csem,
               *, dd, ck, outer, sc_col):
    cid = lax.axis_index("c")
    sid = lax.axis_index("s")
    w = cid * NS + sid
    ebase = w * EPW

    if not sc_col:
        pltpu.async_copy(col_hbm.at[pl.ds(ebase, EPW)], colv, gsem.at[1])

    def zero_body(e, _):
        for j in range(dd // 16):
            gbuf[0, e, pl.ds(j * 16, 16)] = jnp.zeros((16,), jnp.float32)
        return 0

    lax.fori_loop(0, ck, zero_body, 0)
    abase = sid * ROWS_PER_S
    rem = ROWS_PER_S % ck
    for k in range(ROWS_PER_S // ck):
        pltpu.async_copy(gbuf.at[0], acc.at[pl.ds(abase + k * ck, ck)],
                         gsem.at[0])
    pltpu.async_copy(gbuf.at[0, pl.ds(0, rem)],
                     acc.at[pl.ds(abase + ROWS_PER_S - rem, rem)], gsem.at[0])
    for k in range(ROWS_PER_S // ck):
        pltpu.make_async_copy(gbuf.at[0],
                              acc.at[pl.ds(abase + k * ck, ck)],
                              gsem.at[0]).wait()
    pltpu.make_async_copy(gbuf.at[0, pl.ds(0, rem)],
                          acc.at[pl.ds(abase + ROWS_PER_S - rem, rem)],
                          gsem.at[0]).wait()
    if not sc_col:
        pltpu.make_async_copy(col_hbm.at[pl.ds(ebase, EPW)], colv,
                              gsem.at[1]).wait()

    if sc_col:
        for c in range(GPD):
            pltpu.sync_copy(col_hbm.at[pl.ds(ebase + c * ck, ck)],
                            colv.at[c])
        pltpu.async_copy(col_hbm.at[pl.ds(ebase + GPD * ck, ck)],
                         colv.at[GPD], csem.at[GPD])
        for c in range(GPD):
            pltpu.async_copy(h_hbm.at[colv.at[c]], gbuf.at[c], gsem.at[c])
    else:
        for c in range(GPD):
            pltpu.async_copy(h_hbm.at[colv.at[pl.ds(c * ck, ck)]],
                             gbuf.at[c], gsem.at[c])
    for c in range(RPD):
        pltpu.async_copy(row_hbm.at[pl.ds(ebase + c * ck, ck)],
                         rowv.at[c], rsem.at[c])
        pltpu.async_copy(val_hbm.at[pl.ds(ebase + c * ck, ck)],
                         valv.at[c], vsem.at[c])

    plsc.subcore_barrier()

    def outer_body(o, _):
        for b in range(GR):
            s = o * GR + b
            pltpu.make_async_copy(h_hbm.at[pl.ds(0, ck)],
                                  gbuf.at[b], gsem.at[b]).wait()
            pltpu.make_async_copy(val_hbm.at[pl.ds(0, ck)],
                                  valv.at[b], vsem.at[b]).wait()

            def scale_body(g, _, b=b):
                vv = valv[b, pl.ds(g * 16, 16)]
                for l in range(16):
                    v = vv[l]
                    e = g * 16 + l
                    for j in range(dd // 16):
                        gbuf[b, e, pl.ds(j * 16, 16)] = (
                            gbuf[b, e, pl.ds(j * 16, 16)] * v)
                return 0

            lax.fori_loop(0, ck // 16, scale_body, 0)

            pltpu.make_async_copy(row_hbm.at[pl.ds(0, ck)],
                                  rowv.at[b], rsem.at[b]).wait()
            pltpu.async_copy(gbuf.at[b], acc.at[rowv.at[b]], ssem.at[b],
                             add=True)

            nb = (b + GPD) % GR

            def gissue(o=o, b=b, nb=nb):
                t = o * GR + b + GPD
                if sc_col:
                    pltpu.make_async_copy(col_hbm.at[pl.ds(0, ck)],
                                          colv.at[nb], csem.at[nb]).wait()
                    pltpu.async_copy(h_hbm.at[colv.at[nb]],
                                     gbuf.at[nb], gsem.at[nb])
                else:
                    pltpu.async_copy(h_hbm.at[colv.at[pl.ds(t * ck, ck)]],
                                     gbuf.at[nb], gsem.at[nb])

            def cissue(o=o, b=b):
                t = o * GR + b + GPD + 1
                nc = (b + GPD + 1) % GR
                pltpu.async_copy(col_hbm.at[pl.ds(ebase + t * ck, ck)],
                                 colv.at[nc], csem.at[nc])

            def sdrain(nb=nb):
                pltpu.make_async_copy(h_hbm.at[pl.ds(0, ck)],
                                      gbuf.at[nb], ssem.at[nb]).wait()

            def rissue(o=o, b=b):
                t = o * GR + b + RPD
                nr = (b + RPD) % GR
                pltpu.async_copy(row_hbm.at[pl.ds(ebase + t * ck, ck)],
                                 rowv.at[nr], rsem.at[nr])
                pltpu.async_copy(val_hbm.at[pl.ds(ebase + t * ck, ck)],
                                 valv.at[nr], vsem.at[nr])

            if sc_col:
                if b < GR - GPD - 1:
                    cissue()
                else:
                    @pl.when(o < outer - 1)
                    def _(cissue=cissue):
                        cissue()
            if b < GR - GPD:
                @pl.when(o > 0)
                def _(gissue=gissue, sdrain=sdrain):
                    sdrain()
                    gissue()

                @pl.when(o == 0)
                def _(gissue=gissue):
                    gissue()
            else:
                @pl.when(o < outer - 1)
                def _(gissue=gissue, sdrain=sdrain):
                    sdrain()
                    gissue()
            if b < GR - RPD:
                rissue()
            else:
                @pl.when(o < outer - 1)
                def _(rissue=rissue):
                    rissue()
        return 0

    lax.fori_loop(0, outer, outer_body, 0)

    for b in range(GR):
        pltpu.make_async_copy(h_hbm.at[pl.ds(0, ck)],
                              gbuf.at[b], ssem.at[b]).wait()

    plsc.subcore_barrier()

    off = pl.multiple_of(sid * 624, 8)
    pltpu.sync_copy(acc.at[pl.ds(off, 624)], out_hbm.at[cid, pl.ds(off, 624)])

    @pl.when(sid == 0)
    def _():
        pltpu.sync_copy(acc.at[pl.ds(NS * 624, N - NS * 624)],
                        out_hbm.at[cid, pl.ds(NS * 624, N - NS * 624)])


def _make_spmm(dd, ck, sc_col=False):
    outer = EPW // ck // GR
    mesh = plsc.VectorSubcoreMesh(core_axis_name="c", subcore_axis_name="s")
    return pl.kernel(
        functools.partial(_spmm_body, dd=dd, ck=ck, outer=outer,
                          sc_col=sc_col),
        mesh=mesh,
        compiler_params=pltpu.CompilerParams(use_tc_tiling_on_sc=False),
        out_type=jax.ShapeDtypeStruct((NC, N, dd), jnp.float32),
        scratch_types=[
            pltpu.VMEM_SHARED((N, dd), jnp.float32),
            pltpu.VMEM((GR, ck) if sc_col else (EPW,), jnp.int32),
            pltpu.VMEM((GR, ck), jnp.float32),
            pltpu.VMEM((GR, ck), jnp.int32),
            pltpu.VMEM((GR, ck, dd), jnp.float32),
            pltpu.SemaphoreType.DMA((GR,)),
            pltpu.SemaphoreType.DMA((GR,)),
            pltpu.SemaphoreType.DMA((GR,)),
            pltpu.SemaphoreType.DMA((GR,)),
            pltpu.SemaphoreType.DMA((GR,)),
        ],
    )


def _dense1_body(x_ref, w_ref, b_ref, o_ref):
    o_ref[...] = (jnp.dot(x_ref[...], w_ref[...],
                          preferred_element_type=jnp.float32) + b_ref[...])


def _dense2_body(p_ref, w2_ref, b2_ref, wc_ref, o_ref):
    t = jnp.maximum(p_ref[0] + p_ref[1], 0.0)
    h2 = (jnp.dot(t, w2_ref[...], preferred_element_type=jnp.float32)
          + b2_ref[...])
    o_ref[...] = jnp.dot(h2, wc_ref[...], preferred_element_type=jnp.float32)


def _dense3_body(q_ref, bc_ref, o_ref):
    o_ref[...] = (q_ref[0] + q_ref[1])[:, :NCLS] + bc_ref[...]


def kernel(x, edge_index, adj_values, W1, b1, W2, b2, Wc, bc):
    pad = E_PAD - E
    spread = (jnp.arange(pad, dtype=jnp.int32) * 13) % N
    row = jnp.concatenate([edge_index[0].astype(jnp.int32), spread])
    col = jnp.concatenate([edge_index[1].astype(jnp.int32), spread])
    vals = jnp.concatenate(
        [adj_values.astype(jnp.float32), jnp.zeros((pad,), jnp.float32)])

    h = pl.pallas_call(
        _dense1_body,
        out_shape=jax.ShapeDtypeStruct((N, D), jnp.float32),
    )(x, W1, b1.reshape(1, -1))

    p = _make_spmm(D, 64, sc_col=True)(h, row, col, vals)

    Wcp = jnp.pad(Wc, ((0, 0), (0, DC - NCLS)))
    z = pl.pallas_call(
        _dense2_body,
        out_shape=jax.ShapeDtypeStruct((N, DC), jnp.float32),
    )(p, W2, b2.reshape(1, -1), Wcp)

    q = _make_spmm(DC, CHUNK2)(z, row, col, vals)

    return pl.pallas_call(
        _dense3_body,
        out_shape=jax.ShapeDtypeStruct((N, NCLS), jnp.float32),
    )(q, bc.reshape(1, -1))
